# baseline (speedup 1.0000x reference)
import jax
import jax.numpy as jnp
from jax.experimental import pallas as pl
from jax.experimental.pallas import tpu as pltpu

S = 1.0
EPS = 1e-12
SINH_CLIP = 30.0
VAR_FLOOR = 1e-3
BN_EPS = 1e-6
MAX_STEP = 0.5
TOL = 1e-6
N_ITERS = 8

_P = 1
_FMAX = 3.4028235e38
_HI = jax.lax.Precision.HIGHEST


def _asinh(n):
    return jnp.log(n + jnp.sqrt(n * n + 1.0))


def _sinh(a):
    e = jnp.exp(a)
    return 0.5 * (e - 1.0 / e)


def _nan_to_num(v):
    v = jnp.where(jnp.isnan(v), jnp.float32(0.0), v)
    return jnp.clip(v, -_FMAX, _FMAX)


def _row_scalars(xb, m):
    mm = jnp.sum(m * m)
    bu = jax.lax.rsqrt(1.0 + mm)
    a = bu / (1.0 + bu)
    t = jnp.dot(xb, m[:, None], precision=_HI)
    xx = jnp.dot(xb * xb, jnp.ones((xb.shape[1], 1), jnp.float32),
                 precision=_HI)
    q = jnp.sqrt(1.0 + xx) - a * t
    n2 = jnp.maximum(xx + q * (q * mm - 2.0 * t), 0.0)
    return q, n2, t, xx, mm


def _step_kernel(x_ref, aux_ref, out_ref, sig_ref):
    b = pl.program_id(1)
    xb = x_ref[...]
    m = aux_ref[0, :]
    q, n2, _, _, _ = _row_scalars(xb, m)
    n = jnp.sqrt(n2)
    g = _asinh(n) / jnp.maximum(n, EPS)
    gx = jax.lax.dot_general(xb, g, (((0,), (0,)), ((), ())),
                             precision=_HI)
    sg = jnp.sum(g * q)
    contrib = gx.T[None]
    sig = jnp.full((1, 1, 128), sg, dtype=jnp.float32)

    @pl.when(b == 0)
    def _():
        out_ref[...] = contrib
        sig_ref[...] = sig

    @pl.when(b != 0)
    def _():
        out_ref[...] += contrib
        sig_ref[...] += sig


def _var_kernel(x_ref, aux_ref, out_ref):
    b = pl.program_id(1)
    m = aux_ref[0, :]
    _, n2, _, _, _ = _row_scalars(x_ref[...], m)
    d = _asinh(jnp.sqrt(n2))
    s = jnp.sum(d * d)
    contrib = jnp.full((1, 1, 128), s, dtype=jnp.float32)

    @pl.when(b == 0)
    def _():
        out_ref[...] = contrib

    @pl.when(b != 0)
    def _():
        out_ref[...] += contrib


def _xform_kernel(x_ref, aux_ref, out_ref):
    xb = x_ref[...]
    m = aux_ref[0, :]
    w = aux_ref[1, :]
    factor = aux_ref[2, 0]
    gain = aux_ref[3, 0]

    q, n2, t, _, _ = _row_scalars(xb, m)
    n = jnp.sqrt(n2)

    ra = jnp.clip(factor * _asinh(n), -SINH_CLIP, SINH_CLIP)
    c1 = _sinh(ra) / jnp.maximum(n, EPS)

    ww = jnp.sum(w * w)
    bw = jax.lax.rsqrt(1.0 + ww)
    aw = bw / (1.0 + bw)
    tw = jnp.dot(xb, w[:, None], precision=_HI)
    wm = jnp.sum(w * m)
    wz = c1 * (tw - q * wm)
    z2 = c1 * c1 * n2
    r = jnp.sqrt(1.0 + z2) + aw * wz
    x1n2 = jnp.maximum(z2 + r * (2.0 * wz + r * ww), 0.0)

    n1 = jnp.sqrt(x1n2)
    ra1 = jnp.clip(gain * _asinh(n1), -SINH_CLIP, SINH_CLIP)
    c2 = _sinh(ra1) / jnp.maximum(n1, EPS)

    a_coef = c2 * c1
    b_coef = c2 * r
    out_ref[...] = a_coef * xb - (a_coef * q) * m[None, :] + b_coef * w[None, :]


def _expmap0_vec(v):
    n = jnp.linalg.norm(v, axis=-1, keepdims=True)
    coef = jnp.sinh(n) / jnp.maximum(n, EPS)
    return jnp.where(n <= EPS, jnp.zeros_like(v), coef * v)


def _gyro_add_vec(u, v):
    bu = jax.lax.rsqrt(1.0 + jnp.sum(u * u, axis=-1, keepdims=True))
    bv = jax.lax.rsqrt(1.0 + jnp.sum(v * v, axis=-1, keepdims=True))
    coef = (bu / (1.0 + bu)) * jnp.sum(u * v, axis=-1, keepdims=True) \
        + (1.0 - bv) / bv
    return u + v + coef * u


def _make_aux(mean, w_pt=None, factor=None, gain=None):
    aux = jnp.zeros((8, 128), dtype=jnp.float32)
    aux = aux.at[0, :].set(mean)
    if w_pt is not None:
        aux = aux.at[1, :].set(w_pt)
        aux = aux.at[2, 0].set(factor)
        aux = aux.at[3, 0].set(gain)
    return aux


@jax.jit
def kernel(x, weight, shift, post_gain):
    orig_shape = x.shape
    xf = x.reshape(-1, x.shape[-1]).astype(jnp.float32)
    n_rows, d = xf.shape

    blk = 2048 if n_rows % (_P * 2048) == 0 else n_rows // _P
    nb = n_rows // (_P * blk)
    grid = (_P, nb)

    x_spec = pl.BlockSpec((blk, d), lambda p, b: (p * nb + b, 0))
    aux_spec = pl.BlockSpec((8, 128), lambda p, b: (0, 0))
    acc_spec = pl.BlockSpec((1, 1, d), lambda p, b: (p, 0, 0))
    params = pltpu.CompilerParams(
        dimension_semantics=(
            pltpu.GridDimensionSemantics.PARALLEL,
            pltpu.GridDimensionSemantics.ARBITRARY,
        ),
    )

    step_call = pl.pallas_call(
        _step_kernel,
        grid=grid,
        in_specs=[x_spec, aux_spec],
        out_specs=[acc_spec, acc_spec],
        out_shape=[jax.ShapeDtypeStruct((_P, 1, d), jnp.float32),
                   jax.ShapeDtypeStruct((_P, 1, 128), jnp.float32)],
        compiler_params=params,
    )
    var_call = pl.pallas_call(
        _var_kernel,
        grid=grid,
        in_specs=[x_spec, aux_spec],
        out_specs=acc_spec,
        out_shape=jax.ShapeDtypeStruct((_P, 1, 128), jnp.float32),
        compiler_params=params,
    )
    xform_call = pl.pallas_call(
        _xform_kernel,
        grid=grid,
        in_specs=[x_spec, aux_spec],
        out_specs=x_spec,
        out_shape=jax.ShapeDtypeStruct((n_rows, d), jnp.float32),
        compiler_params=params,
    )

    mean = xf[0:1]
    done = jnp.asarray(False)
    for _ in range(N_ITERS):
        psum, sig = step_call(xf, _make_aux(mean[0]))
        gxs = jnp.sum(psum[:, 0], axis=0)
        sgs = jnp.sum(sig[:, 0, 0])
        step = ((gxs - sgs * mean[0]) / n_rows)[None]
        sn = jnp.maximum(jnp.linalg.norm(step), 1e-8)
        step = step * jnp.minimum(MAX_STEP / sn, 1.0)
        new_mean = _gyro_add_vec(mean, _expmap0_vec(step))
        conv = jnp.linalg.norm(new_mean - mean) < TOL
        mean = jnp.where(done, mean, new_mean)
        done = jnp.logical_or(done, conv)
    mean_v = mean[0]

    vpart = var_call(xf, _make_aux(mean_v))
    var = jnp.maximum(_nan_to_num(jnp.sum(vpart[:, 0, 0]) / n_rows), 1e-8)

    w_pt = _expmap0_vec(weight[None, :])[0]
    factor = (shift / jnp.sqrt(jnp.maximum(var, VAR_FLOOR) + BN_EPS))[0]
    gain = jnp.clip(post_gain, 0.5, 3.0)
    out = xform_call(xf, _make_aux(mean_v, w_pt, factor, gain))
    return out.reshape(orig_shape)


# speedup vs baseline: 1.4352x; 1.4352x over previous
import jax
import jax.numpy as jnp
from jax.experimental import pallas as pl
from jax.experimental.pallas import tpu as pltpu

S = 1.0
EPS = 1e-12
SINH_CLIP = 30.0
VAR_FLOOR = 1e-3
BN_EPS = 1e-6
MAX_STEP = 0.5
TOL = 1e-6
N_ITERS = 8

_FMAX = 3.4028235e38
_HI = jax.lax.Precision.HIGHEST


def _asinh(n):
    return jnp.log(n + jnp.sqrt(n * n + 1.0))


def _sinh(a):
    e = jnp.exp(a)
    return 0.5 * (e - 1.0 / e)


def _nan_to_num(v):
    v = jnp.where(jnp.isnan(v), jnp.float32(0.0), v)
    return jnp.clip(v, -_FMAX, _FMAX)


def _row_scalars(xb, m):
    mm = jnp.sum(m * m)
    bu = jax.lax.rsqrt(1.0 + mm)
    a = bu / (1.0 + bu)
    t = jnp.sum(xb * m[None, :], axis=1, keepdims=True)
    xx = jnp.sum(xb * xb, axis=1, keepdims=True)
    q = jnp.sqrt(1.0 + xx) - a * t
    n2 = jnp.maximum(xx + q * (q * mm - 2.0 * t), 0.0)
    return q, n2, t, a


def _frechet_kernel(nb, n_rows, x_ref, aux_ref, mean_out, var_out,
                    mean_s, acc_s, sg_s, var_s, done_s):
    k = pl.program_id(0)
    b = pl.program_id(1)

    @pl.when((k == 0) & (b == 0))
    def _():
        mean_s[...] = aux_ref[0:1, :]
        done_s[0] = 0

    @pl.when(b == 0)
    def _():
        acc_s[...] = jnp.zeros((1, 128), jnp.float32)
        sg_s[...] = jnp.zeros((1, 128), jnp.float32)
        var_s[...] = jnp.zeros((1, 128), jnp.float32)

    xb = x_ref[...]
    m = mean_s[0, :]
    q, n2, _, a = _row_scalars(xb, m)
    n = jnp.sqrt(n2)

    @pl.when(k < N_ITERS)
    def _():
        g = _asinh(n) / jnp.maximum(n, EPS)
        gx = jax.lax.dot_general(xb, g, (((0,), (0,)), ((), ())),
                                 precision=_HI)
        acc_s[...] += gx.T
        sg_s[...] += jnp.full((1, 128), jnp.sum(g * q), jnp.float32)

    @pl.when(k == N_ITERS)
    def _():
        dd = _asinh(n)
        var_s[...] += jnp.full((1, 128), jnp.sum(dd * dd), jnp.float32)

    @pl.when((b == nb - 1) & (k < N_ITERS))
    def _():
        mean = mean_s[0, :]
        sg = sg_s[0, 0]
        step = (acc_s[0, :] - sg * mean) * (1.0 / n_rows)
        sn2 = jnp.sum(step * step)
        sn = jnp.maximum(jnp.sqrt(sn2), 1e-8)
        scale = jnp.minimum(MAX_STEP / sn, 1.0)
        step = step * scale
        ns = jnp.sqrt(sn2) * scale
        ce = _sinh(ns) / jnp.maximum(ns, EPS)
        e = ce * step
        ee = _sinh(ns) * _sinh(ns)
        bv = jax.lax.rsqrt(1.0 + ee)
        me = jnp.sum(mean * e)
        coef = a * me + (1.0 - bv) / bv
        new_mean = mean + e + coef * mean
        dfl = new_mean - mean
        conv = jnp.sum(dfl * dfl) < TOL * TOL
        done = done_s[0] > 0
        mean_s[...] = jnp.where(done, mean, new_mean)[None]
        done_s[0] = jnp.where(done | conv, 1, 0).astype(jnp.int32)

    @pl.when((k == N_ITERS) & (b == nb - 1))
    def _():
        mean_out[...] = mean_s[...][None]
        var_out[...] = var_s[...][None]


def _xform_kernel(x_ref, aux_ref, out_ref):
    xb = x_ref[...]
    m = aux_ref[0, :]
    w = aux_ref[1, :]
    factor = aux_ref[2, 0]
    gain = aux_ref[3, 0]

    q, n2, _, _ = _row_scalars(xb, m)
    n = jnp.sqrt(n2)

    ra = jnp.clip(factor * _asinh(n), -SINH_CLIP, SINH_CLIP)
    c1 = _sinh(ra) / jnp.maximum(n, EPS)

    ww = jnp.sum(w * w)
    bw = jax.lax.rsqrt(1.0 + ww)
    aw = bw / (1.0 + bw)
    tw = jnp.sum(xb * w[None, :], axis=1, keepdims=True)
    wm = jnp.sum(w * m)
    wz = c1 * (tw - q * wm)
    z2 = c1 * c1 * n2
    r = jnp.sqrt(1.0 + z2) + aw * wz
    x1n2 = jnp.maximum(z2 + r * (2.0 * wz + r * ww), 0.0)

    n1 = jnp.sqrt(x1n2)
    ra1 = jnp.clip(gain * _asinh(n1), -SINH_CLIP, SINH_CLIP)
    c2 = _sinh(ra1) / jnp.maximum(n1, EPS)

    a_coef = c2 * c1
    b_coef = c2 * r
    out_ref[...] = a_coef * xb - (a_coef * q) * m[None, :] + b_coef * w[None, :]


def _expmap0_vec(v):
    n = jnp.linalg.norm(v, axis=-1, keepdims=True)
    coef = jnp.sinh(n) / jnp.maximum(n, EPS)
    return jnp.where(n <= EPS, jnp.zeros_like(v), coef * v)


@jax.jit
def kernel(x, weight, shift, post_gain):
    orig_shape = x.shape
    xf = x.reshape(-1, x.shape[-1]).astype(jnp.float32)
    n_rows, d = xf.shape

    blk = 2048 if n_rows % 2048 == 0 else n_rows
    nb = n_rows // blk

    x_spec = pl.BlockSpec((blk, d), lambda k, b: (b, 0))
    aux_spec = pl.BlockSpec((8, 128), lambda k, b: (0, 0))
    fix_spec = pl.BlockSpec((1, 1, d), lambda k, b: (0, 0, 0))
    params = pltpu.CompilerParams(
        dimension_semantics=(
            pltpu.GridDimensionSemantics.ARBITRARY,
            pltpu.GridDimensionSemantics.ARBITRARY,
        ),
    )

    import functools
    frechet_call = pl.pallas_call(
        functools.partial(_frechet_kernel, nb, n_rows),
        grid=(N_ITERS + 1, nb),
        in_specs=[x_spec, aux_spec],
        out_specs=[fix_spec, fix_spec],
        out_shape=[jax.ShapeDtypeStruct((1, 1, d), jnp.float32),
                   jax.ShapeDtypeStruct((1, 1, 128), jnp.float32)],
        scratch_shapes=[
            pltpu.VMEM((1, 128), jnp.float32),
            pltpu.VMEM((1, 128), jnp.float32),
            pltpu.VMEM((1, 128), jnp.float32),
            pltpu.VMEM((1, 128), jnp.float32),
            pltpu.SMEM((1,), jnp.int32),
        ],
        compiler_params=params,
    )
    xform_call = pl.pallas_call(
        _xform_kernel,
        grid=(1, nb),
        in_specs=[x_spec, aux_spec],
        out_specs=x_spec,
        out_shape=jax.ShapeDtypeStruct((n_rows, d), jnp.float32),
        compiler_params=pltpu.CompilerParams(
            dimension_semantics=(
                pltpu.GridDimensionSemantics.PARALLEL,
                pltpu.GridDimensionSemantics.ARBITRARY,
            ),
        ),
    )

    aux0 = jnp.zeros((8, 128), jnp.float32).at[0, :].set(xf[0, :])
    mean_o, var_o = frechet_call(xf, aux0)
    mean_v = mean_o[0, 0]
    var = jnp.maximum(_nan_to_num(var_o[0, 0, 0] / n_rows), 1e-8)

    w_pt = _expmap0_vec(weight[None, :])[0]
    factor = (shift / jnp.sqrt(jnp.maximum(var, VAR_FLOOR) + BN_EPS))[0]
    gain = jnp.clip(post_gain, 0.5, 3.0)
    aux1 = jnp.zeros((8, 128), jnp.float32)
    aux1 = aux1.at[0, :].set(mean_v).at[1, :].set(w_pt)
    aux1 = aux1.at[2, 0].set(factor).at[3, 0].set(gain)
    out = xform_call(xf, aux1)
    return out.reshape(orig_shape)


# speedup vs baseline: 3.9872x; 2.7782x over previous
import functools

import jax
import jax.numpy as jnp
from jax.experimental import pallas as pl
from jax.experimental.pallas import tpu as pltpu

S = 1.0
EPS = 1e-12
SINH_CLIP = 30.0
VAR_FLOOR = 1e-3
BN_EPS = 1e-6
MAX_STEP = 0.5
TOL = 1e-6
N_ITERS = 8

_FMAX = 3.4028235e38


def _asinh(n):
    return jnp.log(n + jnp.sqrt(n * n + 1.0))


def _sinh(a):
    e = jnp.exp(a)
    return 0.5 * (e - 1.0 / e)


def _nan_to_num(v):
    v = jnp.where(jnp.isnan(v), jnp.float32(0.0), v)
    return jnp.clip(v, -_FMAX, _FMAX)


def _mean_scalars(mean_row):
    mm = jnp.sum(mean_row * mean_row)
    bu = jax.lax.rsqrt(1.0 + mm)
    a = bu / (1.0 + bu)
    return mm, a


def _frechet_kernel(nb, n_rows, xt_ref, aux_ref, mean_out, var_out,
                    mean_s, mb_s, xx_s, acc_s, sca_s, done_s):
    k = pl.program_id(0)
    b = pl.program_id(1)

    @pl.when((k == 0) & (b == 0))
    def _():
        mean_s[...] = aux_ref[0:1, :]
        mb_s[...] = jnp.broadcast_to(
            jnp.transpose(aux_ref[0:1, :], (1, 0)), mb_s.shape)
        done_s[0] = 0
        sca_s[0] = 0.0
        sca_s[1] = 0.0

    @pl.when(b == 0)
    def _():
        acc_s[...] = jnp.zeros(acc_s.shape, jnp.float32)

    xtb = xt_ref[...]
    mean_row = mean_s[...][0]
    mm, a = _mean_scalars(mean_row)
    m_b = mb_s[...]

    t = jnp.sum(xtb * m_b, axis=0, keepdims=True)

    @pl.when(k == 0)
    def _():
        xx_s[b] = jnp.sum(xtb * xtb, axis=0, keepdims=True)

    xx = xx_s[b]
    q = jnp.sqrt(1.0 + xx) - a * t
    n2 = jnp.maximum(xx + q * (q * mm - 2.0 * t), 0.0)
    n = jnp.sqrt(n2)

    @pl.when(k < N_ITERS)
    def _():
        g = _asinh(n) / jnp.maximum(n, EPS)
        gxp = xtb * g
        acc_s[:, 0:1] += jnp.sum(gxp, axis=1, keepdims=True)
        sca_s[0] += jnp.sum(g * q)

    @pl.when(k == N_ITERS)
    def _():
        dd = _asinh(n)
        sca_s[1] += jnp.sum(dd * dd)

    @pl.when((b == nb - 1) & (k < N_ITERS))
    def _():
        mean = mean_s[...][0]
        gxs = jnp.transpose(acc_s[:, 0:1], (1, 0))[0]
        sg = sca_s[0]
        step = (gxs - sg * mean) * (1.0 / n_rows)
        sn2 = jnp.sum(step * step)
        sn = jnp.maximum(jnp.sqrt(sn2), 1e-8)
        scale = jnp.minimum(MAX_STEP / sn, 1.0)
        step = step * scale
        ns = jnp.sqrt(sn2) * scale
        ce = _sinh(ns) / jnp.maximum(ns, EPS)
        e = ce * step
        ee = _sinh(ns) * _sinh(ns)
        bv = jax.lax.rsqrt(1.0 + ee)
        me = jnp.sum(mean * e)
        coef = a * me + (1.0 - bv) / bv
        new_mean = mean + e + coef * mean
        dfl = new_mean - mean
        conv = jnp.sum(dfl * dfl) < TOL * TOL
        done = done_s[0] > 0
        kept = jnp.where(done, mean, new_mean)[None]
        mean_s[...] = kept
        mb_s[...] = jnp.broadcast_to(jnp.transpose(kept, (1, 0)), mb_s.shape)
        done_s[0] = jnp.where(done | conv, 1, 0).astype(jnp.int32)
        sca_s[0] = 0.0

    @pl.when((k == N_ITERS) & (b == nb - 1))
    def _():
        mean_out[...] = mean_s[...]
        var_out[...] = jnp.full(var_out.shape, sca_s[1], jnp.float32)


def _xform_kernel(xt_ref, aux_ref, out_ref, mwb_s):
    b = pl.program_id(1)
    xtb = xt_ref[...]
    d = xtb.shape[0]
    mean_row = aux_ref[0, :]
    w_row = aux_ref[1, :]
    factor = aux_ref[2, 0]
    gain = aux_ref[3, 0]

    @pl.when(b == 0)
    def _():
        mwb_s[0:d, :] = jnp.broadcast_to(
            jnp.transpose(aux_ref[0:1, :], (1, 0)), (d, xtb.shape[1]))
        mwb_s[d:2 * d, :] = jnp.broadcast_to(
            jnp.transpose(aux_ref[1:2, :], (1, 0)), (d, xtb.shape[1]))

    m_b = mwb_s[0:d, :]
    w_b = mwb_s[d:2 * d, :]

    mm, a = _mean_scalars(mean_row)
    ww = jnp.sum(w_row * w_row)
    bw = jax.lax.rsqrt(1.0 + ww)
    aw = bw / (1.0 + bw)
    wm = jnp.sum(w_row * mean_row)

    t = jnp.sum(xtb * m_b, axis=0, keepdims=True)
    xx = jnp.sum(xtb * xtb, axis=0, keepdims=True)
    tw = jnp.sum(xtb * w_b, axis=0, keepdims=True)

    q = jnp.sqrt(1.0 + xx) - a * t
    n2 = jnp.maximum(xx + q * (q * mm - 2.0 * t), 0.0)
    n = jnp.sqrt(n2)

    ra = jnp.clip(factor * _asinh(n), -SINH_CLIP, SINH_CLIP)
    c1 = _sinh(ra) / jnp.maximum(n, EPS)

    wz = c1 * (tw - q * wm)
    z2 = c1 * c1 * n2
    r = jnp.sqrt(1.0 + z2) + aw * wz
    x1n2 = jnp.maximum(z2 + r * (2.0 * wz + r * ww), 0.0)

    n1 = jnp.sqrt(x1n2)
    ra1 = jnp.clip(gain * _asinh(n1), -SINH_CLIP, SINH_CLIP)
    c2 = _sinh(ra1) / jnp.maximum(n1, EPS)

    a_coef = c2 * c1
    out_ref[...] = a_coef * xtb - (a_coef * q) * m_b + (c2 * r) * w_b


def _expmap0_vec(v):
    n = jnp.linalg.norm(v, axis=-1, keepdims=True)
    coef = jnp.sinh(n) / jnp.maximum(n, EPS)
    return jnp.where(n <= EPS, jnp.zeros_like(v), coef * v)


@jax.jit
def kernel(x, weight, shift, post_gain):
    orig_shape = x.shape
    xf = x.reshape(-1, x.shape[-1]).astype(jnp.float32)
    n_rows, d = xf.shape
    xt = xf.T

    bc = 2048 if n_rows % 2048 == 0 else n_rows
    nb = n_rows // bc

    xt_spec = pl.BlockSpec((d, bc), lambda k, b: (0, b))
    aux_spec = pl.BlockSpec((8, 128), lambda k, b: (0, 0))
    row_spec = pl.BlockSpec((1, d), lambda k, b: (0, 0))
    params = pltpu.CompilerParams(
        dimension_semantics=(
            pltpu.GridDimensionSemantics.ARBITRARY,
            pltpu.GridDimensionSemantics.ARBITRARY,
        ),
    )

    frechet_call = pl.pallas_call(
        functools.partial(_frechet_kernel, nb, n_rows),
        grid=(N_ITERS + 1, nb),
        in_specs=[xt_spec, aux_spec],
        out_specs=[row_spec, row_spec],
        out_shape=[jax.ShapeDtypeStruct((1, d), jnp.float32),
                   jax.ShapeDtypeStruct((1, 128), jnp.float32)],
        scratch_shapes=[
            pltpu.VMEM((1, d), jnp.float32),
            pltpu.VMEM((d, bc), jnp.float32),
            pltpu.VMEM((nb, 1, bc), jnp.float32),
            pltpu.VMEM((d, 128), jnp.float32),
            pltpu.SMEM((2,), jnp.float32),
            pltpu.SMEM((1,), jnp.int32),
        ],
        compiler_params=params,
    )
    xform_call = pl.pallas_call(
        _xform_kernel,
        grid=(1, nb),
        in_specs=[xt_spec, aux_spec],
        out_specs=xt_spec,
        out_shape=jax.ShapeDtypeStruct((d, n_rows), jnp.float32),
        scratch_shapes=[pltpu.VMEM((2 * d, bc), jnp.float32)],
        compiler_params=pltpu.CompilerParams(
            dimension_semantics=(
                pltpu.GridDimensionSemantics.ARBITRARY,
                pltpu.GridDimensionSemantics.ARBITRARY,
            ),
        ),
    )

    aux0 = jnp.zeros((8, 128), jnp.float32).at[0, :].set(xf[0, :])
    mean_o, var_o = frechet_call(xt, aux0)
    mean_v = mean_o[0]
    var = jnp.maximum(_nan_to_num(var_o[0, 0] / n_rows), 1e-8)

    w_pt = _expmap0_vec(weight[None, :])[0]
    factor = (shift / jnp.sqrt(jnp.maximum(var, VAR_FLOOR) + BN_EPS))[0]
    gain = jnp.clip(post_gain, 0.5, 3.0)
    aux1 = jnp.zeros((8, 128), jnp.float32)
    aux1 = aux1.at[0, :].set(mean_v).at[1, :].set(w_pt)
    aux1 = aux1.at[2, 0].set(factor).at[3, 0].set(gain)
    out_t = xform_call(xt, aux1)
    return out_t.T.reshape(orig_shape)
